# baseline (speedup 1.0000x reference)
import functools

import jax
import jax.numpy as jnp
from jax import lax
from jax.experimental import pallas as pl
from jax.experimental.pallas import tpu as pltpu
from jax.experimental.pallas import tpu_sc as plsc

NUM_USERS = 100000
NUM_ITEMS = 100000
D = 128
N_CAT = 26
CAT_VOCAB = 1000
CAT_D = 16
N_NUM = 13
B = 4096
H1 = 2048
H2 = 1024
CAT_DIM = N_CAT * CAT_D

NC = 2
NS = 16
NW = NC * NS

NCHUNK = 1
NB = B // NCHUNK
BPW = NB // NW
CPW = NB * N_CAT // 128 // NW

CGRP = (8, 8, 8, 2)

BT = 512


def _sc_gather_body(uidx, iidx, cidx, ut, it, ct,
                    uo, io, co0, co1, co2, co3,
                    uidx_v, iidx_v, cidx_v, urows, irows, crows,
                    usem, isem, csem):
    wid = lax.axis_index("s") * NC + lax.axis_index("c")
    base = wid * BPW
    su = pltpu.async_copy(uidx.at[pl.ds(base, BPW)], uidx_v, usem)
    si = pltpu.async_copy(iidx.at[pl.ds(base, BPW)], iidx_v, isem)
    off = 0
    voff = 0
    nbytes = 0
    for g in CGRP:
        n = g * BPW
        pltpu.async_copy(cidx.at[pl.ds(off + wid * n, n)],
                         cidx_v.at[pl.ds(voff, n)], csem)
        off += NB * g
        voff += n
        nbytes += n
    su.wait()
    gu = pltpu.async_copy(ut.at[uidx_v], urows, usem)
    si.wait()
    gi = pltpu.async_copy(it.at[iidx_v], irows, isem)
    pltpu.make_async_copy(cidx.at[pl.ds(0, voff)], cidx_v, csem).wait()

    def fire(j, carry):
        pltpu.async_copy(ct.at[cidx_v.at[pl.ds(j * 128, 128)]],
                         crows.at[j], csem)
        return carry

    lax.fori_loop(0, CPW, fire, 0)
    gu.wait()
    wu = pltpu.async_copy(urows, uo.at[pl.ds(base, BPW)], usem)
    gi.wait()
    wi = pltpu.async_copy(irows, io.at[pl.ds(base, BPW)], isem)
    for g, co in zip(CGRP, (co0, co1, co2, co3)):
        pltpu.make_async_copy(co.at[wid], crows.at[pl.ds(0, g)], csem).wait()
    r = 0
    wcs = []
    for g, co in zip(CGRP, (co0, co1, co2, co3)):
        wcs.append(pltpu.async_copy(crows.at[pl.ds(r, g)], co.at[wid], csem))
        r += g
    wu.wait()
    wi.wait()
    for wc in wcs:
        wc.wait()


def _sc_gather(uidx, iidx, cidx, ut, it, ct):
    mesh = plsc.VectorSubcoreMesh(
        core_axis_name="c", subcore_axis_name="s",
        num_cores=NC, num_subcores=NS)
    f = pl.kernel(
        _sc_gather_body,
        out_type=(
            jax.ShapeDtypeStruct((NB, D), jnp.float32),
            jax.ShapeDtypeStruct((NB, D), jnp.float32),
            jax.ShapeDtypeStruct((NW, CGRP[0], 128, CAT_D), jnp.float32),
            jax.ShapeDtypeStruct((NW, CGRP[1], 128, CAT_D), jnp.float32),
            jax.ShapeDtypeStruct((NW, CGRP[2], 128, CAT_D), jnp.float32),
            jax.ShapeDtypeStruct((NW, CGRP[3], 128, CAT_D), jnp.float32),
        ),
        mesh=mesh,
        scratch_types=(
            pltpu.VMEM((BPW,), jnp.int32),
            pltpu.VMEM((BPW,), jnp.int32),
            pltpu.VMEM((CPW * 128,), jnp.int32),
            pltpu.VMEM((BPW, D), jnp.float32),
            pltpu.VMEM((BPW, D), jnp.float32),
            pltpu.VMEM((CPW, 128, CAT_D), jnp.float32),
            pltpu.SemaphoreType.DMA,
            pltpu.SemaphoreType.DMA,
            pltpu.SemaphoreType.DMA,
        ),
        compiler_params=pltpu.CompilerParams(use_tc_tiling_on_sc=False),
    )
    return f(uidx, iidx, cidx, ut, it, ct)


def _mlp_body(u, i, c0, c1, c2, c3, n, w1, b1, w2, b2, w3, b3, out):
    bf = jnp.bfloat16
    h = jnp.dot(u[...].astype(bf), w1[pl.ds(0, D), :].astype(bf),
                preferred_element_type=jnp.float32)
    h += jnp.dot(i[...].astype(bf), w1[pl.ds(D, D), :].astype(bf),
                 preferred_element_type=jnp.float32)
    r = 2 * D
    for cref, g in zip((c0, c1, c2, c3), CGRP):
        w = g * CAT_D
        h += jnp.dot(cref[...].astype(bf), w1[pl.ds(r, w), :].astype(bf),
                     preferred_element_type=jnp.float32)
        r += w
    h += jnp.dot(n[...], w1[pl.ds(r, N_NUM), :],
                 preferred_element_type=jnp.float32)
    h = jnp.maximum(h + b1[...], 0.0).astype(bf)
    h = jnp.maximum(
        jnp.dot(h, w2[...].astype(bf), preferred_element_type=jnp.float32)
        + b2[...], 0.0)
    out[...] = jnp.dot(h, w3[...], preferred_element_type=jnp.float32) + b3[0, 0]


def _mlp(u, i, c0, c1, c2, c3, n, w1, b1, w2, b2, w3, b3):
    bspec = lambda r, cdim: pl.BlockSpec((r, cdim), lambda b: (b, 0))
    wspec = lambda r, cdim: pl.BlockSpec((r, cdim), lambda b: (0, 0))
    return pl.pallas_call(
        _mlp_body,
        grid=(NB // BT,),
        in_specs=[
            bspec(BT, D), bspec(BT, D),
            bspec(BT, CGRP[0] * CAT_D), bspec(BT, CGRP[1] * CAT_D),
            bspec(BT, CGRP[2] * CAT_D), bspec(BT, CGRP[3] * CAT_D),
            bspec(BT, N_NUM),
            wspec(D + D + CAT_DIM + N_NUM, H1),
            wspec(1, H1), wspec(H1, H2), wspec(1, H2), wspec(H2, 1),
            wspec(1, 1),
        ],
        out_specs=pl.BlockSpec((BT, 1), lambda b: (b, 0)),
        out_shape=jax.ShapeDtypeStruct((NB, 1), jnp.float32),
        compiler_params=pltpu.CompilerParams(
            dimension_semantics=("arbitrary",),
            vmem_limit_bytes=100 * 1024 * 1024,
        ),
    )(u, i, c0, c1, c2, c3, n, w1, b1, w2, b2, w3, b3)


def kernel(x, category_dict, numeric_dict, user_table, item_table, cat_table,
           W1, b1, W2, b2, W3, b3):
    uidx = x[:, 0].astype(jnp.int32)
    iidx = x[:, 1].astype(jnp.int32)
    offs = jnp.arange(N_CAT, dtype=jnp.int32) * CAT_VOCAB
    cd = category_dict.astype(jnp.int32) + offs[None, :]
    parts = []
    lo = 0
    for g in CGRP:
        parts.append(cd[:, lo:lo + g].reshape(-1))
        lo += g
    cidx = jnp.concatenate(parts)
    ue, ie, ce0, ce1, ce2, ce3 = _sc_gather(uidx, iidx, cidx,
                                            user_table, item_table, cat_table)
    cats = [ce.reshape(NB, g * CAT_D) for ce, g in
            zip((ce0, ce1, ce2, ce3), CGRP)]
    return _mlp(ue, ie, *cats, numeric_dict,
                W1, b1.reshape(1, H1), W2, b2.reshape(1, H2), W3,
                b3.reshape(1, 1))


# speedup vs baseline: 1.0425x; 1.0425x over previous
import functools

import jax
import jax.numpy as jnp
from jax import lax
from jax.experimental import pallas as pl
from jax.experimental.pallas import tpu as pltpu
from jax.experimental.pallas import tpu_sc as plsc

NUM_USERS = 100000
NUM_ITEMS = 100000
D = 128
N_CAT = 26
CAT_VOCAB = 1000
CAT_D = 16
N_NUM = 13
B = 4096
H1 = 2048
H2 = 1024
CAT_DIM = N_CAT * CAT_D

NC = 2
NS = 16
NW = NC * NS

NCHUNK = 1
NB = B // NCHUNK
BPW = NB // NW
CPW = NB * N_CAT // 128 // NW

BT = 1024


def _sc_gather_body(uidx, iidx, cidx, ut, it, ct, uo, io, co,
                    uidx_v, iidx_v, cidx_v, urows, irows, crows,
                    usem, isem, csem):
    wid = lax.axis_index("s") * NC + lax.axis_index("c")
    base = wid * BPW
    su = pltpu.async_copy(uidx.at[pl.ds(base, BPW)], uidx_v, usem)
    si = pltpu.async_copy(iidx.at[pl.ds(base, BPW)], iidx_v, isem)
    sc = pltpu.async_copy(cidx.at[pl.ds(wid * CPW * 128, CPW * 128)],
                          cidx_v, csem)
    su.wait()
    gu = pltpu.async_copy(ut.at[uidx_v], urows, usem)
    si.wait()
    gi = pltpu.async_copy(it.at[iidx_v], irows, isem)
    sc.wait()

    def fire(j, carry):
        pltpu.async_copy(ct.at[cidx_v.at[pl.ds(j * 128, 128)]],
                         crows.at[j], csem)
        return carry

    lax.fori_loop(0, CPW, fire, 0)
    gu.wait()
    wu = pltpu.async_copy(urows, uo.at[pl.ds(base, BPW)], usem)
    gi.wait()
    wi = pltpu.async_copy(irows, io.at[pl.ds(base, BPW)], isem)
    pltpu.make_async_copy(co.at[wid], crows, csem).wait()
    wc = pltpu.async_copy(crows, co.at[wid], csem)
    wu.wait()
    wi.wait()
    wc.wait()


def _sc_gather(uidx, iidx, cidx, ut, it, ct):
    mesh = plsc.VectorSubcoreMesh(
        core_axis_name="c", subcore_axis_name="s",
        num_cores=NC, num_subcores=NS)
    f = pl.kernel(
        _sc_gather_body,
        out_type=(
            jax.ShapeDtypeStruct((NB, D), jnp.float32),
            jax.ShapeDtypeStruct((NB, D), jnp.float32),
            jax.ShapeDtypeStruct((NW, CPW, 128, CAT_D), jnp.float32),
        ),
        mesh=mesh,
        scratch_types=(
            pltpu.VMEM((BPW,), jnp.int32),
            pltpu.VMEM((BPW,), jnp.int32),
            pltpu.VMEM((CPW * 128,), jnp.int32),
            pltpu.VMEM((BPW, D), jnp.float32),
            pltpu.VMEM((BPW, D), jnp.float32),
            pltpu.VMEM((CPW, 128, CAT_D), jnp.float32),
            pltpu.SemaphoreType.DMA,
            pltpu.SemaphoreType.DMA,
            pltpu.SemaphoreType.DMA,
        ),
        compiler_params=pltpu.CompilerParams(use_tc_tiling_on_sc=False),
    )
    return f(uidx, iidx, cidx, ut, it, ct)


def _mlp_body(u, i, c, n, w1, b1, w2, b2, w3, b3, out):
    bf = jnp.bfloat16
    h = jnp.dot(u[...].astype(bf), w1[pl.ds(0, D), :].astype(bf),
                preferred_element_type=jnp.float32)
    h += jnp.dot(i[...].astype(bf), w1[pl.ds(D, D), :].astype(bf),
                 preferred_element_type=jnp.float32)
    h += jnp.dot(c[...].astype(bf), w1[pl.ds(2 * D, CAT_DIM), :].astype(bf),
                 preferred_element_type=jnp.float32)
    h += jnp.dot(n[...], w1[pl.ds(2 * D + CAT_DIM, N_NUM), :],
                 preferred_element_type=jnp.float32)
    h = jnp.maximum(h + b1[...], 0.0).astype(bf)
    h = jnp.maximum(
        jnp.dot(h, w2[...].astype(bf), preferred_element_type=jnp.float32)
        + b2[...], 0.0)
    out[...] = jnp.dot(h, w3[...], preferred_element_type=jnp.float32) + b3[0, 0]


def _mlp(u, i, c, n, w1, b1, w2, b2, w3, b3):
    bspec = lambda r, cdim: pl.BlockSpec((r, cdim), lambda b: (b, 0))
    wspec = lambda r, cdim: pl.BlockSpec((r, cdim), lambda b: (0, 0))
    return pl.pallas_call(
        _mlp_body,
        grid=(NB // BT,),
        in_specs=[
            bspec(BT, D), bspec(BT, D), bspec(BT, CAT_DIM), bspec(BT, N_NUM),
            wspec(D + D + CAT_DIM + N_NUM, H1),
            wspec(1, H1), wspec(H1, H2), wspec(1, H2), wspec(H2, 1),
            wspec(1, 1),
        ],
        out_specs=pl.BlockSpec((BT, 1), lambda b: (b, 0)),
        out_shape=jax.ShapeDtypeStruct((NB, 1), jnp.float32),
        compiler_params=pltpu.CompilerParams(
            dimension_semantics=("arbitrary",),
            vmem_limit_bytes=100 * 1024 * 1024,
        ),
    )(u, i, c, n, w1, b1, w2, b2, w3, b3)


def kernel(x, category_dict, numeric_dict, user_table, item_table, cat_table,
           W1, b1, W2, b2, W3, b3):
    uidx = x[:, 0].astype(jnp.int32)
    iidx = x[:, 1].astype(jnp.int32)
    offs = jnp.arange(N_CAT, dtype=jnp.int32) * CAT_VOCAB
    cidx = (category_dict.astype(jnp.int32) + offs[None, :]).reshape(-1)
    ue, ie, ce = _sc_gather(uidx, iidx, cidx,
                            user_table, item_table, cat_table)
    return _mlp(ue, ie, ce.reshape(NB, CAT_DIM), numeric_dict,
                W1, b1.reshape(1, H1), W2, b2.reshape(1, H2), W3,
                b3.reshape(1, 1))
